# baseline (speedup 1.0000x reference)
import jax
import jax.numpy as jnp
from jax.experimental import pallas as pl

EMBD = 128
SLOPE = 0.01


def _leaky(x):
    return jnp.where(x >= 0, x, SLOPE * x)


def _emb_kern(x_ref, w0_ref, b0_ref, w1_ref, b1_ref, o_ref):
    y = jnp.dot(x_ref[...], w0_ref[...], preferred_element_type=jnp.float32)
    y = y + b0_ref[...]
    y = jnp.dot(y, w1_ref[...], preferred_element_type=jnp.float32)
    y = y + b1_ref[...]
    o_ref[...] = _leaky(y)


def _lin_kern(x_ref, w_ref, b_ref, o_ref):
    o_ref[...] = (
        jnp.dot(x_ref[...], w_ref[...], preferred_element_type=jnp.float32)
        + b_ref[...]
    )


def _msg_kern(ed_ref, lg_ref, rg_ref, we_ref, be_ref, wf_ref, bf_ref, o_ref):
    e = jnp.dot(ed_ref[...], we_ref[...], preferred_element_type=jnp.float32)
    e = jax.nn.sigmoid(e + be_ref[...])
    t = lg_ref[...] + e * rg_ref[...]
    o_ref[...] = (
        jnp.dot(t, wf_ref[...], preferred_element_type=jnp.float32) + bf_ref[...]
    )


def _convout_kern(s_ref, inv_ref, right_ref, w0a_ref, w0b_ref, b0_ref,
                  w1_ref, b1_ref, scale_ref, o_ref):
    agg = s_ref[...] * inv_ref[...]
    y = jnp.dot(agg, w0a_ref[...], preferred_element_type=jnp.float32)
    y = y + jnp.dot(right_ref[...], w0b_ref[...], preferred_element_type=jnp.float32)
    y = y + b0_ref[...]
    y = jnp.dot(y, w1_ref[...], preferred_element_type=jnp.float32) + b1_ref[...]
    rms = jnp.sqrt(jnp.mean(y * y, axis=-1, keepdims=True) + 1e-8)
    y = scale_ref[0, 0] * (y / rms)
    o_ref[...] = _leaky(y)


def _head_kern(x_ref, w0_ref, b0_ref, w1_ref, o_ref):
    y = jnp.dot(x_ref[...], w0_ref[...], preferred_element_type=jnp.float32)
    y = y + b0_ref[...]
    o_ref[...] = jnp.dot(y, w1_ref[...], preferred_element_type=jnp.float32)


_BN = 10000


def _full(shape):
    return pl.BlockSpec(shape, lambda i: (0, 0))


def _emb_mlp(x8, w0t, b0, w1t, b1):
    n = x8.shape[0]
    return pl.pallas_call(
        _emb_kern,
        grid=(n // _BN,),
        in_specs=[
            pl.BlockSpec((_BN, 8), lambda i: (i, 0)),
            _full((8, EMBD)), _full((1, EMBD)),
            _full((EMBD, EMBD)), _full((1, EMBD)),
        ],
        out_specs=pl.BlockSpec((_BN, EMBD), lambda i: (i, 0)),
        out_shape=jax.ShapeDtypeStruct((n, EMBD), jnp.float32),
    )(x8, w0t, b0, w1t, b1)


def _linear(x, wt, b):
    n = x.shape[0]
    return pl.pallas_call(
        _lin_kern,
        grid=(n // _BN,),
        in_specs=[
            pl.BlockSpec((_BN, EMBD), lambda i: (i, 0)),
            _full((EMBD, EMBD)), _full((1, EMBD)),
        ],
        out_specs=pl.BlockSpec((_BN, EMBD), lambda i: (i, 0)),
        out_shape=jax.ShapeDtypeStruct((n, EMBD), jnp.float32),
    )(x, wt, b)


def _messages(ed, lg, rg, wet, be, wft, bf):
    n = ed.shape[0]
    return pl.pallas_call(
        _msg_kern,
        grid=(n // _BN,),
        in_specs=[
            pl.BlockSpec((_BN, EMBD), lambda i: (i, 0)),
            pl.BlockSpec((_BN, EMBD), lambda i: (i, 0)),
            pl.BlockSpec((_BN, EMBD), lambda i: (i, 0)),
            _full((EMBD, EMBD)), _full((1, EMBD)),
            _full((EMBD, EMBD)), _full((1, EMBD)),
        ],
        out_specs=pl.BlockSpec((_BN, EMBD), lambda i: (i, 0)),
        out_shape=jax.ShapeDtypeStruct((n, EMBD), jnp.float32),
    )(ed, lg, rg, wet, be, wft, bf)


def _conv_out(summed, inv, right, w0at, w0bt, b0, w1t, b1, scale):
    n = summed.shape[0]
    return pl.pallas_call(
        _convout_kern,
        grid=(n // _BN,),
        in_specs=[
            pl.BlockSpec((_BN, EMBD), lambda i: (i, 0)),
            pl.BlockSpec((_BN, 1), lambda i: (i, 0)),
            pl.BlockSpec((_BN, EMBD), lambda i: (i, 0)),
            _full((EMBD, EMBD)), _full((EMBD, EMBD)), _full((1, EMBD)),
            _full((EMBD, EMBD)), _full((1, EMBD)),
            pl.BlockSpec((1, 1), lambda i: (0, 0)),
        ],
        out_specs=pl.BlockSpec((_BN, EMBD), lambda i: (i, 0)),
        out_shape=jax.ShapeDtypeStruct((n, EMBD), jnp.float32),
    )(summed, inv, right, w0at, w0bt, b0, w1t, b1, scale)


def _head(x, w0t, b0, w1t_pad):
    n = x.shape[0]
    return pl.pallas_call(
        _head_kern,
        grid=(n // _BN,),
        in_specs=[
            pl.BlockSpec((_BN, EMBD), lambda i: (i, 0)),
            _full((EMBD, EMBD)), _full((1, EMBD)),
            _full((EMBD, EMBD)),
        ],
        out_specs=pl.BlockSpec((_BN, EMBD), lambda i: (i, 0)),
        out_shape=jax.ShapeDtypeStruct((n, EMBD), jnp.float32),
    )(x, w0t, b0, w1t_pad)


def _wt(lp):
    return lp["W"].T


def _b2(lp):
    return lp["b"].reshape(1, -1)


def _conv_layer(cp, left, gather_left_idx, gather_right_idx, ed, right,
                seg_ids, inv_cnt):
    l_all = _linear(right, _wt(cp["left"]), _b2(cp["left"]))
    r_all = _linear(left, _wt(cp["right"]), _b2(cp["right"]))
    lg = jnp.take(l_all, gather_left_idx, axis=0)
    rg = jnp.take(r_all, gather_right_idx, axis=0)
    msg = _messages(ed, lg, rg, _wt(cp["edge"]), _b2(cp["edge"]),
                    _wt(cp["final"]), _b2(cp["final"]))
    summed = jax.ops.segment_sum(msg, seg_ids, num_segments=right.shape[0])
    w0 = cp["out0"]["W"]
    return _conv_out(summed, inv_cnt, right,
                     w0[:, :EMBD].T, w0[:, EMBD:].T, _b2(cp["out0"]),
                     _wt(cp["out1"]), _b2(cp["out1"]),
                     cp["scale"].reshape(1, 1))


def kernel(item_features, edge_indices, edge_features, column_features, params):
    p = params
    src = edge_indices[0]
    dst = edge_indices[1]
    n_items = item_features.shape[0]
    n_cols = column_features.shape[0]
    n_edges = edge_features.shape[0]

    def pad8(x):
        return jnp.pad(x, ((0, 0), (0, 8 - x.shape[1])))

    def padw8(w):
        wt = w.T
        return jnp.pad(wt, ((0, 8 - wt.shape[0]), (0, 0)))

    it = _emb_mlp(pad8(item_features), padw8(p["item_emb0"]["W"]),
                  _b2(p["item_emb0"]), _wt(p["item_emb1"]), _b2(p["item_emb1"]))
    ed = _emb_mlp(pad8(edge_features), padw8(p["edge_emb0"]["W"]),
                  _b2(p["edge_emb0"]), _wt(p["edge_emb1"]), _b2(p["edge_emb1"]))
    col = _emb_mlp(pad8(column_features), padw8(p["col_emb0"]["W"]),
                   _b2(p["col_emb0"]), _wt(p["col_emb1"]), _b2(p["col_emb1"]))

    ones = jnp.ones((n_edges,), jnp.float32)
    cnt_i = jax.ops.segment_sum(ones, src, num_segments=n_items)
    cnt_c = jax.ops.segment_sum(ones, dst, num_segments=n_cols)
    inv_i = (1.0 / jnp.maximum(cnt_i, 1.0)).reshape(-1, 1)
    inv_c = (1.0 / jnp.maximum(cnt_c, 1.0)).reshape(-1, 1)

    it = _conv_layer(p["c2i_1"], col, src, dst, ed, it, src, inv_i)
    col = _conv_layer(p["i2c_1"], it, dst, src, ed, col, dst, inv_c)
    it = _linear(it, _wt(p["item_between"]), _b2(p["item_between"]))
    col = _linear(col, _wt(p["col_between"]), _b2(p["col_between"]))
    it = _conv_layer(p["c2i_2"], col, src, dst, ed, it, src, inv_i)
    col = _conv_layer(p["i2c_2"], it, dst, src, ed, col, dst, inv_c)
    it = _linear(it, _wt(p["item_between"]), _b2(p["item_between"]))
    col = _linear(col, _wt(p["col_between"]), _b2(p["col_between"]))
    it = _conv_layer(p["c2i_3"], col, src, dst, ed, it, src, inv_i)
    col = _conv_layer(p["i2c_3"], it, dst, src, ed, col, dst, inv_c)

    w1pad = jnp.pad(p["head1"]["W"].T, ((0, 0), (0, EMBD - 1)))
    out = _head(col, _wt(p["head0"]), _b2(p["head0"]), w1pad)
    return out[:, 0]
